# Initial kernel scaffold; baseline (speedup 1.0000x reference)
import jax
import jax.numpy as jnp
from jax.experimental import pallas as pl


def kernel(x, convert_w, convert_b, c1_0_w1, c1_0_b1, c1_0_w2, c1_0_b2, c1_1_w1, c1_1_b1, c1_1_w2, c1_1_b2, c2_0_w1, c2_0_b1, c2_0_w2, c2_0_b2, c2_1_w1, c2_1_b1, c2_1_w2, c2_1_b2, c3_0_w1, c3_0_b1, c3_0_w2, c3_0_b2, c3_1_w1, c3_1_b1, c3_1_w2, c3_1_b2, c4_0_w1, c4_0_b1, c4_0_w2, c4_0_b2, c4_1_w1, c4_1_b1, c4_1_w2, c4_1_b2, c5_0_w1, c5_0_b1, c5_0_w2, c5_0_b2, c5_1_w1, c5_1_b1, c5_1_w2, c5_1_b2, cat1_w, cat1_b, cat2_w, cat2_b, out1_w, out1_b, out2_w, out2_b):
    raise NotImplementedError("write your pallas kernel here")


# speedup vs baseline: 1.2901x; 1.2901x over previous
import functools

import jax
import jax.numpy as jnp
from jax.experimental import pallas as pl
from jax.experimental.pallas import tpu as pltpu

_VMEM_LIMIT = 52 * 1024 * 1024
_VMEM_BUDGET = 32 * 1024 * 1024
_CAT_ROWS = 4096


def _rup(x, m):
    return (x + m - 1) // m * m


def _row_bytes(W, C):
    return _rup(W + 2, 8) * _rup(C, 128) * 4


def _pair_footprint(tH, W, C):
    rb = _row_bytes(W, C)
    slab = 2 * (tH + 8) * rb
    pads = (3 * tH + 12) * rb
    patch = (tH + 6) * W * _rup(9 * C, 128) * 2
    acc = (tH + 6) * W * _rup(C, 128) * 4
    outb = 2 * tH * _rup(W, 8) * _rup(C, 128) * 4
    return slab + pads + patch + acc + outb


def _head_footprint(tH, W, Cin, Cout):
    rb = _row_bytes(W, Cin)
    slab = 2 * (tH + 2) * rb
    patch = tH * W * _rup(9 * Cin, 128) * 2
    acc = 2 * tH * W * _rup(Cout, 128) * 4
    outb = 2 * tH * _rup(W, 8) * _rup(Cout, 128) * 4
    return slab + patch + acc + outb


def _pick_th(H, halo, fp, even=False):
    cands = [d for d in range(1, H + 1)
             if H % d == 0 and (d >= halo or d == H) and (not even or d % 2 == 0)]
    fits = [d for d in cands if fp(d) <= _VMEM_BUDGET]
    return max(fits) if fits else min(cands)


def _lrelu(x):
    return jnp.where(x > 0, x, 0.2 * x)


def _strip_copies(x_hbm, xs_ref, sem, n, i, slot, tH, halo, H, W):
    r0 = i * tH
    interior = pltpu.make_async_copy(
        x_hbm.at[n, pl.ds(r0, tH), :, :],
        xs_ref.at[slot, pl.ds(halo, tH), pl.ds(1, W), :],
        sem.at[slot, 0])
    top = pltpu.make_async_copy(
        x_hbm.at[n, pl.ds(jnp.maximum(r0 - halo, 0), halo), :, :],
        xs_ref.at[slot, pl.ds(0, halo), pl.ds(1, W), :],
        sem.at[slot, 1])
    bot = pltpu.make_async_copy(
        x_hbm.at[n, pl.ds(jnp.minimum(r0 + tH, max(H - halo, 0)), halo), :, :],
        xs_ref.at[slot, pl.ds(halo + tH, halo), pl.ds(1, W), :],
        sem.at[slot, 2])
    return interior, top, bot


def _strip_start(x_hbm, xs_ref, sem, n, i, slot, nH, tH, halo, H, W):
    interior, top, bot = _strip_copies(x_hbm, xs_ref, sem, n, i, slot, tH, halo, H, W)
    interior.start()

    @pl.when(i > 0)
    def _():
        top.start()

    @pl.when(i < nH - 1)
    def _():
        bot.start()


def _strip_wait(x_hbm, xs_ref, sem, n, i, slot, nH, tH, halo, H, W):
    interior, top, bot = _strip_copies(x_hbm, xs_ref, sem, n, i, slot, tH, halo, H, W)
    interior.wait()

    @pl.when(i > 0)
    def _():
        top.wait()

    @pl.when(i < nH - 1)
    def _():
        bot.wait()


def _strip_fetch(x_hbm, xs_ref, sem, n, i, slot, nH, tH, halo, H, W):
    @pl.when(i == 0)
    def _():
        _strip_start(x_hbm, xs_ref, sem, n, i, 0, nH, tH, halo, H, W)

    @pl.when(i + 1 < nH)
    def _():
        _strip_start(x_hbm, xs_ref, sem, n, i + 1, (i + 1) % 2, nH, tH, halo, H, W)

    _strip_wait(x_hbm, xs_ref, sem, n, i, slot, nH, tH, halo, H, W)


def _zero_slab_edges(xs_ref, i, nH, tH, halo, W, C):
    R = tH + 2 * halo
    last = nH - 1

    @pl.when(i == 0)
    def _():
        zc = jnp.zeros((R, 1, C), jnp.float32)
        xs_ref[0, :, 0:1, :] = zc
        xs_ref[0, :, W + 1:W + 2, :] = zc
        xs_ref[0, 0:halo, :, :] = jnp.zeros((halo, W + 2, C), jnp.float32)

    if nH > 1:
        @pl.when(i == 1)
        def _():
            zc = jnp.zeros((R, 1, C), jnp.float32)
            xs_ref[1, :, 0:1, :] = zc
            xs_ref[1, :, W + 1:W + 2, :] = zc

    @pl.when(i == last)
    def _():
        xs_ref[last % 2, halo + tH:R, :, :] = jnp.zeros((halo, W + 2, C), jnp.float32)


def _conv9(xpad_bf, w_ref, bias, out_h, out_w):
    cin = xpad_bf.shape[-1]
    m = out_h * out_w
    taps = [xpad_bf[dy:dy + out_h, dx:dx + out_w, :].reshape(m, cin)
            for dy in range(3) for dx in range(3)]
    patch = jnp.concatenate(taps, axis=-1)
    return jnp.dot(patch, w_ref[...], preferred_element_type=jnp.float32) + bias


def _stage_pad(pad_ref, val, i, last, w, top_zero, bot_zero, C):
    rows = val.shape[0]
    pad_ref[:, 1:1 + w, :] = val

    @pl.when(i == 0)
    def _():
        pad_ref[0:top_zero, :, :] = jnp.zeros((top_zero, w + 2, C), jnp.float32)

    @pl.when(i == last)
    def _():
        pad_ref[rows - bot_zero:rows, :, :] = jnp.zeros((bot_zero, w + 2, C), jnp.float32)


def _pair_kernel(x_hbm, w1a, b1a, w2a, b2a, w1b, b1b, w2b, b2b,
                 o_ref, xs_ref, y1p, m1p, y2p, sem, *, H, tH, nH):
    n, i = pl.program_id(0), pl.program_id(1)
    halo = 4
    W = o_ref.shape[2]
    C = o_ref.shape[3]
    slot = i % 2
    last = nH - 1

    _strip_fetch(x_hbm, xs_ref, sem, n, i, slot, nH, tH, halo, H, W)
    _zero_slab_edges(xs_ref, i, nH, tH, halo, W, C)

    @pl.when(i == 0)
    def _():
        for ref, rows in ((y1p, tH + 6), (m1p, tH + 4), (y2p, tH + 2)):
            zc = jnp.zeros((rows, 1, C), jnp.float32)
            ref[:, 0:1, :] = zc
            ref[:, W + 1:W + 2, :] = zc

    xv = xs_ref[slot]
    xbf = xv.astype(jnp.bfloat16)

    y1 = _lrelu(_conv9(xbf, w1a, b1a[...], tH + 6, W)).reshape(tH + 6, W, C)
    _stage_pad(y1p, y1, i, last, W, 3, 3, C)

    z = _lrelu(_conv9(y1p[...].astype(jnp.bfloat16), w2a, b2a[...], tH + 4, W))
    m1 = z.reshape(tH + 4, W, C) + xv[2:2 + tH + 4, 1:1 + W, :]
    _stage_pad(m1p, m1, i, last, W, 2, 2, C)

    y2 = _lrelu(_conv9(m1p[...].astype(jnp.bfloat16), w1b, b1b[...], tH + 2, W))
    _stage_pad(y2p, y2.reshape(tH + 2, W, C), i, last, W, 1, 1, C)

    z2 = _lrelu(_conv9(y2p[...].astype(jnp.bfloat16), w2b, b2b[...], tH, W))
    out = z2.reshape(tH, W, C) + m1p[2:2 + tH, 1:1 + W, :]
    o_ref[0] = out


def _msrb_pair(x, pa, pb):
    N, H, W, C = x.shape
    halo = 4
    tH = _pick_th(H, halo, lambda t: _pair_footprint(t, W, C))
    nH = H // tH

    def prep(wb):
        (w1, b1), (w2, b2) = wb
        return (w1.reshape(9 * C, C).astype(jnp.bfloat16), b1.reshape(1, C),
                w2.reshape(9 * C, C).astype(jnp.bfloat16), b2.reshape(1, C))

    wspec = pl.BlockSpec((9 * C, C), lambda n, i: (0, 0))
    bspec = pl.BlockSpec((1, C), lambda n, i: (0, 0))

    cost = pl.CostEstimate(
        flops=4 * N * (H + 4 * nH) * W * 9 * C * C * 2,
        transcendentals=0,
        bytes_accessed=4 * (N * (H + 2 * halo * nH) * W * C + N * H * W * C))

    res = pl.pallas_call(
        functools.partial(_pair_kernel, H=H, tH=tH, nH=nH),
        out_shape=jax.ShapeDtypeStruct((N, H, W, C), jnp.float32),
        grid=(N, nH),
        in_specs=[pl.BlockSpec(memory_space=pl.ANY)]
                 + [wspec, bspec, wspec, bspec] * 2,
        out_specs=pl.BlockSpec((1, tH, W, C), lambda n, i: (n, i, 0, 0)),
        scratch_shapes=[pltpu.VMEM((2, tH + 2 * halo, W + 2, C), jnp.float32),
                        pltpu.VMEM((tH + 6, W + 2, C), jnp.float32),
                        pltpu.VMEM((tH + 4, W + 2, C), jnp.float32),
                        pltpu.VMEM((tH + 2, W + 2, C), jnp.float32),
                        pltpu.SemaphoreType.DMA((2, 3))],
        compiler_params=pltpu.CompilerParams(
            dimension_semantics=("parallel", "arbitrary"),
            vmem_limit_bytes=_VMEM_LIMIT),
        cost_estimate=cost,
    )(x, *prep(pa), *prep(pb))
    return res


def _conv3_kernel(x_hbm, w_ref, b_ref, o_ref, xs_ref, sem, *, H, tH, nH):
    n, i = pl.program_id(0), pl.program_id(1)
    halo = 1
    W = o_ref.shape[2]
    Cin = xs_ref.shape[3]
    Cout = o_ref.shape[3]
    slot = i % 2

    _strip_fetch(x_hbm, xs_ref, sem, n, i, slot, nH, tH, halo, H, W)
    _zero_slab_edges(xs_ref, i, nH, tH, halo, W, Cin)

    xbf = xs_ref[slot].astype(jnp.bfloat16)
    y = _lrelu(_conv9(xbf, w_ref, b_ref[...], tH, W))
    o_ref[0] = y.reshape(tH, W, Cout)


def _conv3(x, w, b):
    N, H, W, Cin = x.shape
    Cout = w.shape[-1]
    tH = _pick_th(H, 1, lambda t: _head_footprint(t, W, Cin, Cout))
    nH = H // tH
    cost = pl.CostEstimate(
        flops=2 * N * H * W * 9 * Cin * Cout, transcendentals=0,
        bytes_accessed=4 * (N * H * W * (Cin + Cout)))
    return pl.pallas_call(
        functools.partial(_conv3_kernel, H=H, tH=tH, nH=nH),
        out_shape=jax.ShapeDtypeStruct((N, H, W, Cout), jnp.float32),
        grid=(N, nH),
        in_specs=[pl.BlockSpec(memory_space=pl.ANY),
                  pl.BlockSpec((9 * Cin, Cout), lambda n, i: (0, 0)),
                  pl.BlockSpec((1, Cout), lambda n, i: (0, 0))],
        out_specs=pl.BlockSpec((1, tH, W, Cout), lambda n, i: (n, i, 0, 0)),
        scratch_shapes=[pltpu.VMEM((2, tH + 2, W + 2, Cin), jnp.float32),
                        pltpu.SemaphoreType.DMA((2, 3))],
        compiler_params=pltpu.CompilerParams(
            dimension_semantics=("parallel", "arbitrary"),
            vmem_limit_bytes=_VMEM_LIMIT),
        cost_estimate=cost,
    )(x, w.reshape(9 * Cin, Cout).astype(jnp.bfloat16), b.reshape(1, Cout))


def _out_kernel(x_hbm, w1_ref, b1_ref, w2_ref, b2_ref, o_ref, xs_ref, sem,
                *, H, tH, nH):
    n, i = pl.program_id(0), pl.program_id(1)
    halo = 1
    W = o_ref.shape[2]
    Cin = xs_ref.shape[3]
    Cout = o_ref.shape[3]
    slot = i % 2

    _strip_fetch(x_hbm, xs_ref, sem, n, i, slot, nH, tH, halo, H, W)
    _zero_slab_edges(xs_ref, i, nH, tH, halo, W, Cin)

    xbf = xs_ref[slot].astype(jnp.bfloat16)
    y = _lrelu(_conv9(xbf, w1_ref, b1_ref[...], tH, W))
    z = jnp.dot(y.astype(jnp.bfloat16), w2_ref[...],
                preferred_element_type=jnp.float32) + b2_ref[...]
    o_ref[0] = z.reshape(tH, W, Cout)


def _out_head(x, w1, b1, w2, b2):
    N, H, W, Cin = x.shape
    Cmid = w1.shape[-1]
    Cout = w2.shape[-1]
    tH = _pick_th(H, 1, lambda t: _head_footprint(t, W, Cin, Cmid))
    nH = H // tH
    cost = pl.CostEstimate(
        flops=2 * N * H * W * (9 * Cin * Cmid + Cmid * Cout), transcendentals=0,
        bytes_accessed=4 * (N * H * W * (Cin + Cout)))
    return pl.pallas_call(
        functools.partial(_out_kernel, H=H, tH=tH, nH=nH),
        out_shape=jax.ShapeDtypeStruct((N, H, W, Cout), jnp.float32),
        grid=(N, nH),
        in_specs=[pl.BlockSpec(memory_space=pl.ANY),
                  pl.BlockSpec((9 * Cin, Cmid), lambda n, i: (0, 0)),
                  pl.BlockSpec((1, Cmid), lambda n, i: (0, 0)),
                  pl.BlockSpec((Cmid, Cout), lambda n, i: (0, 0)),
                  pl.BlockSpec((1, Cout), lambda n, i: (0, 0))],
        out_specs=pl.BlockSpec((1, tH, W, Cout), lambda n, i: (n, i, 0, 0)),
        scratch_shapes=[pltpu.VMEM((2, tH + 2, W + 2, Cin), jnp.float32),
                        pltpu.SemaphoreType.DMA((2, 3))],
        compiler_params=pltpu.CompilerParams(
            dimension_semantics=("parallel", "arbitrary"),
            vmem_limit_bytes=_VMEM_LIMIT),
        cost_estimate=cost,
    )(x, w1.reshape(9 * Cin, Cmid).astype(jnp.bfloat16), b1.reshape(1, Cmid),
      w2.reshape(Cmid, Cout).astype(jnp.bfloat16), b2.reshape(1, Cout))


def _cat_kernel(a_ref, b_ref, w_ref, bias_ref, o_ref):
    ab = jnp.concatenate([a_ref[...], b_ref[...]], axis=-1).astype(jnp.bfloat16)
    y = jnp.dot(ab, w_ref[...], preferred_element_type=jnp.float32) + bias_ref[...]
    o_ref[...] = _lrelu(y)


def _cat_conv(a, b, w4d, bias):
    n, h, w_, mid = a.shape
    wfull = w4d.reshape(2 * mid, -1).astype(jnp.bfloat16)
    cout = wfull.shape[1]
    a2 = a.reshape(-1, mid)
    b2 = b.reshape(-1, mid)
    M = a2.shape[0]
    tm = min(M, _CAT_ROWS)
    cost = pl.CostEstimate(
        flops=2 * M * 2 * mid * cout, transcendentals=0,
        bytes_accessed=4 * (M * (2 * mid + cout)))
    out = pl.pallas_call(
        _cat_kernel,
        out_shape=jax.ShapeDtypeStruct((M, cout), jnp.float32),
        grid=(pl.cdiv(M, tm),),
        in_specs=[pl.BlockSpec((tm, mid), lambda i: (i, 0)),
                  pl.BlockSpec((tm, mid), lambda i: (i, 0)),
                  pl.BlockSpec((2 * mid, cout), lambda i: (0, 0)),
                  pl.BlockSpec((1, cout), lambda i: (0, 0))],
        out_specs=pl.BlockSpec((tm, cout), lambda i: (i, 0)),
        compiler_params=pltpu.CompilerParams(
            dimension_semantics=("parallel",),
            vmem_limit_bytes=_VMEM_LIMIT),
        cost_estimate=cost,
    )(a2, b2, wfull, bias.reshape(1, cout))
    return out.reshape(n, h, w_, cout)


def _maxpool2(x):
    n, h, w, c = x.shape
    return x.reshape(n, h // 2, 2, w // 2, 2, c).max(axis=(2, 4))


def _upsample2(x, out_h, out_w):
    n, h, w, c = x.shape

    def coords(out_s, in_s):
        if out_s == 1 or in_s == 1:
            return jnp.zeros((out_s,), jnp.float32)
        return jnp.arange(out_s, dtype=jnp.float32) * ((in_s - 1) / (out_s - 1))

    ys, xs = coords(out_h, h), coords(out_w, w)
    y0 = jnp.floor(ys).astype(jnp.int32)
    x0 = jnp.floor(xs).astype(jnp.int32)
    y1 = jnp.minimum(y0 + 1, h - 1)
    x1 = jnp.minimum(x0 + 1, w - 1)
    wy = (ys - y0.astype(jnp.float32))[None, :, None, None]
    wx = (xs - x0.astype(jnp.float32))[None, None, :, None]

    def g(yi, xi):
        return x[:, yi][:, :, xi]

    top = g(y0, x0) * (1 - wx) + g(y0, x1) * wx
    bot = g(y1, x0) * (1 - wx) + g(y1, x1) * wx
    return top * (1 - wy) + bot * wy


@jax.jit
def kernel(x, convert_w, convert_b,
           c1_0_w1, c1_0_b1, c1_0_w2, c1_0_b2, c1_1_w1, c1_1_b1, c1_1_w2, c1_1_b2,
           c2_0_w1, c2_0_b1, c2_0_w2, c2_0_b2, c2_1_w1, c2_1_b1, c2_1_w2, c2_1_b2,
           c3_0_w1, c3_0_b1, c3_0_w2, c3_0_b2, c3_1_w1, c3_1_b1, c3_1_w2, c3_1_b2,
           c4_0_w1, c4_0_b1, c4_0_w2, c4_0_b2, c4_1_w1, c4_1_b1, c4_1_w2, c4_1_b2,
           c5_0_w1, c5_0_b1, c5_0_w2, c5_0_b2, c5_1_w1, c5_1_b1, c5_1_w2, c5_1_b2,
           cat1_w, cat1_b, cat2_w, cat2_b, out1_w, out1_b, out2_w, out2_b):
    convert = _conv3(x, convert_w, convert_b)

    conv1 = _msrb_pair(convert,
                       ((c1_0_w1, c1_0_b1), (c1_0_w2, c1_0_b2)),
                       ((c1_1_w1, c1_1_b1), (c1_1_w2, c1_1_b2)))
    pool1 = _maxpool2(conv1)
    conv2 = _msrb_pair(pool1,
                       ((c2_0_w1, c2_0_b1), (c2_0_w2, c2_0_b2)),
                       ((c2_1_w1, c2_1_b1), (c2_1_w2, c2_1_b2)))
    pool2 = _maxpool2(conv2)
    conv3 = _msrb_pair(pool2,
                       ((c3_0_w1, c3_0_b1), (c3_0_w2, c3_0_b2)),
                       ((c3_1_w1, c3_1_b1), (c3_1_w2, c3_1_b2)))

    h2, w2_ = conv2.shape[1], conv2.shape[2]
    up3 = _upsample2(conv3, h2, w2_)
    conv4 = _msrb_pair(_cat_conv(up3, conv2, cat1_w, cat1_b),
                       ((c4_0_w1, c4_0_b1), (c4_0_w2, c4_0_b2)),
                       ((c4_1_w1, c4_1_b1), (c4_1_w2, c4_1_b2)))

    h1, w1_ = conv1.shape[1], conv1.shape[2]
    up4 = _upsample2(conv4, h1, w1_)
    conv5 = _msrb_pair(_cat_conv(up4, conv1, cat2_w, cat2_b),
                       ((c5_0_w1, c5_0_b1), (c5_0_w2, c5_0_b2)),
                       ((c5_1_w1, c5_1_b1), (c5_1_w2, c5_1_b2)))

    return _out_head(conv5, out1_w, out1_b, out2_w, out2_b)
